# baseline (speedup 1.0000x reference)
import functools

import jax
import jax.numpy as jnp
from jax.experimental import pallas as pl
from jax.experimental.pallas import tpu as pltpu


def _fused_body(W, x_ref, w1t_ref, wdw_ref, bias_ref, o_ref):
    HW = x_ref.shape[1]
    Cout = w1t_ref.shape[1]
    M = HW // 2
    j = pl.program_id(1)
    wdw = wdw_ref[...]
    bias = bias_ref[...]

    def half(lo, top_edge):
        xm = x_ref[0, lo:lo + M + W]
        y = jnp.dot(xm, w1t_ref[...],
                    preferred_element_type=jnp.float32)

        row = jax.lax.broadcasted_iota(jnp.int32, (M + W, Cout), 0)
        w_in_row = row & (W - 1)
        l = pltpu.roll(y, shift=1, axis=0)
        l = jnp.where(w_in_row > 0, l, 0.0)
        r = pltpu.roll(y, shift=M + W - 1, axis=0)
        r = jnp.where(w_in_row < W - 1, r, 0.0)

        def trow(di):
            return (l * wdw[3 * di + 0:3 * di + 1, :]
                    + y * wdw[3 * di + 1:3 * di + 2, :]
                    + r * wdw[3 * di + 2:3 * di + 3, :])

        zrows = jnp.zeros((W, Cout), jnp.float32)
        if top_edge:
            tm = jnp.concatenate([zrows, trow(0)[:M - W]], axis=0)
            tp = trow(2)[W:M + W]
            acc = trow(1)[:M] + tm + tp
        else:
            tm = trow(0)[:M]
            tp = jnp.concatenate([trow(2)[2 * W:M + W], zrows], axis=0)
            acc = trow(1)[W:M + W] + tm + tp
        o_ref[0] = jnp.maximum(acc + bias, 0.0)

    @pl.when(j == 0)
    def _():
        half(0, True)

    @pl.when(j == 1)
    def _():
        half(M - W, False)


def kernel(x_nchw, w1, wdw, gamma, beta, run_mean, run_var):
    eps = 1e-5
    N, Cin, H, W = x_nchw.shape
    Cout = w1.shape[0]
    HW = H * W
    f32 = jnp.float32

    inv = (gamma.astype(f32) / jnp.sqrt(run_var.astype(f32) + eps))
    bias = (beta.astype(f32) - run_mean.astype(f32) * inv)

    x2 = jnp.transpose(x_nchw, (0, 2, 3, 1)).reshape(N, HW, Cin)
    w1t = jnp.transpose(w1.astype(f32), (1, 0))
    wdw_k = (wdw.astype(f32) * inv[:, None, None]).reshape(Cout, 9)
    wdw_k = jnp.transpose(wdw_k, (1, 0))
    bias_k = bias[None, :]

    flops = 2 * N * HW * Cin * Cout + 19 * N * HW * Cout
    bytes_accessed = 4 * (x2.size + w1t.size + wdw_k.size + bias_k.size
                          + N * Cout * HW)

    out = pl.pallas_call(
        functools.partial(_fused_body, W),
        out_shape=jax.ShapeDtypeStruct((N, HW, Cout), f32),
        grid=(N, 2),
        in_specs=[
            pl.BlockSpec((1, HW, Cin), lambda n, j: (n, 0, 0)),
            pl.BlockSpec((Cin, Cout), lambda n, j: (0, 0)),
            pl.BlockSpec((9, Cout), lambda n, j: (0, 0)),
            pl.BlockSpec((1, Cout), lambda n, j: (0, 0)),
        ],
        out_specs=pl.BlockSpec((1, HW // 2, Cout), lambda n, j: (n, j, 0)),
        compiler_params=pltpu.CompilerParams(
            dimension_semantics=("parallel", "arbitrary"),
            vmem_limit_bytes=100 * 1024 * 1024,
        ),
        cost_estimate=pl.CostEstimate(
            flops=flops, transcendentals=0, bytes_accessed=bytes_accessed),
    )(x2, w1t, wdw_k, bias_k)

    return jnp.transpose(out.reshape(N, H, W, Cout), (0, 3, 1, 2))


# speedup vs baseline: 1.2980x; 1.2980x over previous
import functools

import jax
import jax.numpy as jnp
from jax.experimental import pallas as pl
from jax.experimental.pallas import tpu as pltpu


def _fused_body(W, x_ref, w1t_ref, wdw_ref, bias_ref, o_ref):
    HW = x_ref.shape[1]
    Cout = w1t_ref.shape[1]

    y = jnp.dot(x_ref[0], w1t_ref[...],
                preferred_element_type=jnp.float32)

    row = jax.lax.broadcasted_iota(jnp.int32, (HW, Cout), 0)
    w_in_row = row & (W - 1)
    l = pltpu.roll(y, shift=1, axis=0)
    l = jnp.where(w_in_row > 0, l, 0.0)
    r = pltpu.roll(y, shift=HW - 1, axis=0)
    r = jnp.where(w_in_row < W - 1, r, 0.0)

    wdw = wdw_ref[...]

    def trow(di):
        return (l * wdw[3 * di + 0:3 * di + 1, :]
                + y * wdw[3 * di + 1:3 * di + 2, :]
                + r * wdw[3 * di + 2:3 * di + 3, :])

    tm = pltpu.roll(trow(0), shift=W, axis=0)
    tp = pltpu.roll(trow(2), shift=HW - W, axis=0)
    acc = (trow(1)
           + jnp.where(row >= W, tm, 0.0)
           + jnp.where(row < HW - W, tp, 0.0))

    o_ref[0] = jnp.maximum(acc + bias_ref[...], 0.0)


def kernel(x_nchw, w1, wdw, gamma, beta, run_mean, run_var):
    eps = 1e-5
    N, Cin, H, W = x_nchw.shape
    Cout = w1.shape[0]
    HW = H * W
    f32 = jnp.float32

    inv = (gamma.astype(f32) / jnp.sqrt(run_var.astype(f32) + eps))
    bias = (beta.astype(f32) - run_mean.astype(f32) * inv)

    x2 = jnp.transpose(x_nchw, (0, 2, 3, 1)).reshape(N, HW, Cin)
    w1t = jnp.transpose(w1.astype(f32), (1, 0))
    wdw_k = (wdw.astype(f32) * inv[:, None, None]).reshape(Cout, 9)
    wdw_k = jnp.transpose(wdw_k, (1, 0))
    bias_k = bias[None, :]

    flops = 2 * N * HW * Cin * Cout + 19 * N * HW * Cout
    bytes_accessed = 4 * (x2.size + w1t.size + wdw_k.size + bias_k.size
                          + N * Cout * HW)

    out = pl.pallas_call(
        functools.partial(_fused_body, W),
        out_shape=jax.ShapeDtypeStruct((N, HW, Cout), f32),
        grid=(N,),
        in_specs=[
            pl.BlockSpec((1, HW, Cin), lambda n: (n, 0, 0)),
            pl.BlockSpec((Cin, Cout), lambda n: (0, 0)),
            pl.BlockSpec((9, Cout), lambda n: (0, 0)),
            pl.BlockSpec((1, Cout), lambda n: (0, 0)),
        ],
        out_specs=pl.BlockSpec((1, HW, Cout), lambda n: (n, 0, 0)),
        compiler_params=pltpu.CompilerParams(
            dimension_semantics=("parallel",),
            vmem_limit_bytes=100 * 1024 * 1024,
        ),
        cost_estimate=pl.CostEstimate(
            flops=flops, transcendentals=0, bytes_accessed=bytes_accessed),
    )(x2, w1t, wdw_k, bias_k)

    return jnp.transpose(out.reshape(N, H, W, Cout), (0, 3, 1, 2))
